# Initial kernel scaffold; baseline (speedup 1.0000x reference)
import jax
import jax.numpy as jnp
from jax.experimental import pallas as pl


def kernel(daytime, W_day, W_time, W_node):
    raise NotImplementedError("write your pallas kernel here")


# speedup vs baseline: 1.8510x; 1.8510x over previous
import jax
import jax.numpy as jnp
from jax import lax
from jax.experimental import pallas as pl
from jax.experimental.pallas import tpu as pltpu
from jax.experimental.pallas import tpu_sc as plsc

NC, NS, L = 2, 16, 16
NW = NC * NS
B, T, N = 32, 12, 1024
BT = B * T
DAY_SIZE, TIME_SIZE, NODE_SIZE = 32, 32, 64
C = DAY_SIZE + TIME_SIZE + NODE_SIZE
NPW = N // NW
IDX_CHUNK = 128
N_CHUNKS = BT // IDX_CHUNK


def _sc_body(didx_hbm, tidx_hbm, wday_hbm, wtime_hbm, wnode_hbm, out_hbm,
             didx_v, tidx_v, drows_v, trows_v, node_v, buf0, buf1,
             sem_g, sem0, sem1):
    cid = lax.axis_index("c")
    sid = lax.axis_index("s")
    wid = sid * NC + cid
    n0 = wid * NPW

    pltpu.sync_copy(didx_hbm, didx_v)
    pltpu.sync_copy(tidx_hbm, tidx_v)

    for j in range(N_CHUNKS):
        pltpu.async_copy(
            wday_hbm.at[didx_v.at[j]],
            drows_v.at[pl.ds(j * IDX_CHUNK, IDX_CHUNK), :], sem_g).wait()
        pltpu.async_copy(
            wtime_hbm.at[tidx_v.at[j]],
            trows_v.at[pl.ds(j * IDX_CHUNK, IDX_CHUNK), :], sem_g).wait()

    pltpu.sync_copy(wnode_hbm.at[pl.ds(n0, NPW)], node_v)
    for buf in (buf0, buf1):
        for r in range(NPW):
            for j in range(NODE_SIZE // L):
                buf[r, pl.ds(NODE_SIZE + j * L, L)] = node_v[r, pl.ds(j * L, L)]

    def fill_and_send(bt, buf, sem):
        d0 = drows_v[bt, pl.ds(0, L)]
        d1 = drows_v[bt, pl.ds(L, L)]
        t0 = trows_v[bt, pl.ds(0, L)]
        t1 = trows_v[bt, pl.ds(L, L)]
        for r in range(NPW):
            buf[r, pl.ds(0, L)] = d0
            buf[r, pl.ds(L, L)] = d1
            buf[r, pl.ds(2 * L, L)] = t0
            buf[r, pl.ds(3 * L, L)] = t1
        pltpu.async_copy(buf, out_hbm.at[bt, pl.ds(n0, NPW), :], sem)

    def wait_prev(buf, sem):
        pltpu.make_async_copy(buf, out_hbm.at[0, pl.ds(n0, NPW), :], sem).wait()

    fill_and_send(0, buf0, sem0)
    fill_and_send(1, buf1, sem1)

    def body(i, carry):
        bt = i * 2
        wait_prev(buf0, sem0)
        fill_and_send(bt, buf0, sem0)
        wait_prev(buf1, sem1)
        fill_and_send(bt + 1, buf1, sem1)
        return carry

    lax.fori_loop(1, BT // 2, body, 0)
    wait_prev(buf0, sem0)
    wait_prev(buf1, sem1)


def kernel(daytime, W_day, W_time, W_node):
    dt = daytime.astype(jnp.int32)
    day_idx = dt[..., 0].reshape(N_CHUNKS, IDX_CHUNK)
    time_idx = dt[..., 1].reshape(N_CHUNKS, IDX_CHUNK)

    mesh = plsc.VectorSubcoreMesh(core_axis_name="c", subcore_axis_name="s",
                                  num_cores=NC, num_subcores=NS)
    out = pl.kernel(
        _sc_body,
        out_type=jax.ShapeDtypeStruct((BT, N, C), jnp.float32),
        mesh=mesh,
        compiler_params=pltpu.CompilerParams(use_tc_tiling_on_sc=False),
        scratch_types=[
            pltpu.VMEM((N_CHUNKS, IDX_CHUNK), jnp.int32),
            pltpu.VMEM((N_CHUNKS, IDX_CHUNK), jnp.int32),
            pltpu.VMEM((BT, DAY_SIZE), jnp.float32),
            pltpu.VMEM((BT, TIME_SIZE), jnp.float32),
            pltpu.VMEM((NPW, NODE_SIZE), jnp.float32),
            pltpu.VMEM((NPW, C), jnp.float32),
            pltpu.VMEM((NPW, C), jnp.float32),
            pltpu.SemaphoreType.DMA,
            pltpu.SemaphoreType.DMA,
            pltpu.SemaphoreType.DMA,
        ],
    )(day_idx, time_idx, W_day, W_time, W_node)
    return out.reshape(B, T, N, C)


# speedup vs baseline: 1.8946x; 1.0236x over previous
import jax
import jax.numpy as jnp
from jax import lax
from jax.experimental import pallas as pl
from jax.experimental.pallas import tpu as pltpu
from jax.experimental.pallas import tpu_sc as plsc

NC, NS, L = 2, 16, 16
NW = NC * NS
B, T, N = 32, 12, 1024
BT = B * T
DAY_SIZE, TIME_SIZE, NODE_SIZE = 32, 32, 64
C = DAY_SIZE + TIME_SIZE + NODE_SIZE
NPW = N // NW
IDX_CHUNK = 128
N_CHUNKS = BT // IDX_CHUNK
K = 8


def _sc_body(didx_hbm, tidx_hbm, wday_hbm, wtime_hbm, wnode_hbm, out_hbm,
             didx_v, tidx_v, drows_v, trows_v, node_v, buf0, buf1,
             sem_g, sem0, sem1):
    cid = lax.axis_index("c")
    sid = lax.axis_index("s")
    wid = sid * NC + cid
    n0 = wid * NPW

    pltpu.sync_copy(didx_hbm, didx_v)
    pltpu.sync_copy(tidx_hbm, tidx_v)

    for j in range(N_CHUNKS):
        pltpu.async_copy(
            wday_hbm.at[didx_v.at[j]],
            drows_v.at[pl.ds(j * IDX_CHUNK, IDX_CHUNK), :], sem_g).wait()
        pltpu.async_copy(
            wtime_hbm.at[tidx_v.at[j]],
            trows_v.at[pl.ds(j * IDX_CHUNK, IDX_CHUNK), :], sem_g).wait()

    pltpu.sync_copy(wnode_hbm.at[pl.ds(n0, NPW)], node_v)
    for buf in (buf0, buf1):
        def init_k(k, carry):
            for r in range(NPW):
                for j in range(NODE_SIZE // L):
                    buf[k, r, pl.ds(NODE_SIZE + j * L, L)] = \
                        node_v[r, pl.ds(j * L, L)]
            return carry
        lax.fori_loop(0, K, init_k, 0)

    def fill_and_send(bt0, buf, sem):
        def fill_k(k, carry):
            bt = bt0 + k
            d0 = drows_v[bt, pl.ds(0, L)]
            d1 = drows_v[bt, pl.ds(L, L)]
            t0 = trows_v[bt, pl.ds(0, L)]
            t1 = trows_v[bt, pl.ds(L, L)]
            for r in range(NPW):
                buf[k, r, pl.ds(0, L)] = d0
                buf[k, r, pl.ds(L, L)] = d1
                buf[k, r, pl.ds(2 * L, L)] = t0
                buf[k, r, pl.ds(3 * L, L)] = t1
            return carry
        lax.fori_loop(0, K, fill_k, 0)
        pltpu.async_copy(buf, out_hbm.at[pl.ds(bt0, K), pl.ds(n0, NPW), :], sem)

    def wait_prev(buf, sem):
        pltpu.make_async_copy(
            buf, out_hbm.at[pl.ds(0, K), pl.ds(n0, NPW), :], sem).wait()

    fill_and_send(0, buf0, sem0)
    fill_and_send(K, buf1, sem1)

    def body(i, carry):
        bt0 = i * 2 * K
        wait_prev(buf0, sem0)
        fill_and_send(bt0, buf0, sem0)
        wait_prev(buf1, sem1)
        fill_and_send(bt0 + K, buf1, sem1)
        return carry

    lax.fori_loop(1, BT // (2 * K), body, 0)
    wait_prev(buf0, sem0)
    wait_prev(buf1, sem1)


def kernel(daytime, W_day, W_time, W_node):
    dt = daytime.astype(jnp.int32)
    day_idx = dt[..., 0].reshape(N_CHUNKS, IDX_CHUNK)
    time_idx = dt[..., 1].reshape(N_CHUNKS, IDX_CHUNK)

    mesh = plsc.VectorSubcoreMesh(core_axis_name="c", subcore_axis_name="s",
                                  num_cores=NC, num_subcores=NS)
    out = pl.kernel(
        _sc_body,
        out_type=jax.ShapeDtypeStruct((BT, N, C), jnp.float32),
        mesh=mesh,
        compiler_params=pltpu.CompilerParams(use_tc_tiling_on_sc=False),
        scratch_types=[
            pltpu.VMEM((N_CHUNKS, IDX_CHUNK), jnp.int32),
            pltpu.VMEM((N_CHUNKS, IDX_CHUNK), jnp.int32),
            pltpu.VMEM((BT, DAY_SIZE), jnp.float32),
            pltpu.VMEM((BT, TIME_SIZE), jnp.float32),
            pltpu.VMEM((NPW, NODE_SIZE), jnp.float32),
            pltpu.VMEM((K, NPW, C), jnp.float32),
            pltpu.VMEM((K, NPW, C), jnp.float32),
            pltpu.SemaphoreType.DMA,
            pltpu.SemaphoreType.DMA,
            pltpu.SemaphoreType.DMA,
        ],
    )(day_idx, time_idx, W_day, W_time, W_node)
    return out.reshape(B, T, N, C)
